# baseline (speedup 1.0000x reference)
import functools

import jax
import jax.numpy as jnp
from jax import lax
from jax.experimental import pallas as pl
from jax.experimental.pallas import tpu as pltpu
from jax.experimental.pallas import tpu_sc as plsc

_N = 50000
_D = 512
_G = 128
_R = 80
_NB = _N // _R
_NC = 2
_NS = 16
_NW = _NC * _NS
_TRIPS = (_NB + _NW - 1) // _NW
_ACC = _G * _D


def _pool_sc(x_hbm, batch_hbm, part_hbm, rows_v, idx_v, accf_v, ld_v,
             red_v, stage_sh):
    cid = lax.axis_index("c")
    sid = lax.axis_index("s")
    wid = cid * _NS + sid

    def _zbody(i, carry):
        for k in range(8):
            accf_v[pl.ds(i * 128 + k * 16, 16)] = jnp.zeros((16,),
                                                            jnp.float32)
        return carry

    lax.fori_loop(0, _ACC // 128, _zbody, 0)

    lanes = lax.broadcasted_iota(jnp.int32, (16,), 0)

    def _body(t, carry):
        blk = wid + t * _NW

        @pl.when(blk < _NB)
        def _():
            base = blk * _R
            pltpu.sync_copy(batch_hbm.at[pl.ds(base, _R)], idx_v)
            pltpu.sync_copy(x_hbm.at[pl.ds(base, _R)], rows_v)

            @plsc.parallel_loop(0, _R // 16, 1)
            def _grp(g):
                ids16 = idx_v[pl.ds(g * 16, 16)]
                first = ids16[0]
                last = ids16[15]

                @pl.when(first == last)
                def _fast():
                    seg_vec = plsc.load_gather(
                        idx_v, [jnp.full((16,), g * 16, jnp.int32)])
                    tgt = seg_vec * _D + lanes
                    for c in range(_D // 16):
                        vs = [rows_v[g * 16 + rr, pl.ds(c * 16, 16)]
                              for rr in range(16)]
                        while len(vs) > 1:
                            vs = [vs[i] + vs[i + 1]
                                  for i in range(0, len(vs) - 1, 2)] + (
                                      [vs[-1]] if len(vs) % 2 else [])
                        plsc.addupdate_scatter(accf_v, [tgt + (c * 16)],
                                               vs[0])

                @pl.when(first != last)
                def _slow():
                    for rr in range(16):
                        seg_vec = plsc.load_gather(
                            idx_v, [jnp.full((16,), g * 16 + rr, jnp.int32)])
                        tgt = seg_vec * _D + lanes
                        for c in range(_D // 16):
                            vals = rows_v[g * 16 + rr, pl.ds(c * 16, 16)]
                            plsc.addupdate_scatter(accf_v, [tgt + (c * 16)],
                                                   vals)

        return carry

    lax.fori_loop(0, _TRIPS, _body, 0)

    def _round(q, carry):
        pltpu.sync_copy(accf_v.at[pl.ds(q * 16 * _D, 16 * _D)],
                        stage_sh.at[sid])
        plsc.subcore_barrier()

        pltpu.sync_copy(stage_sh.at[0, pl.ds(sid * _D, _D)], red_v)

        def _rbody(src, carry2):
            pltpu.sync_copy(stage_sh.at[src, pl.ds(sid * _D, _D)], ld_v)

            def _abody(i, carry3):
                o = i * 16
                red_v[pl.ds(o, 16)] = red_v[pl.ds(o, 16)] + ld_v[pl.ds(o, 16)]
                return carry3

            lax.fori_loop(0, _D // 16, _abody, 0)
            return carry2

        lax.fori_loop(1, _NS, _rbody, 0)

        pltpu.sync_copy(red_v, part_hbm.at[cid * _G + q * 16 + sid])
        plsc.subcore_barrier()
        return carry

    lax.fori_loop(0, _G // 16, _round, 0)


_pool = pl.kernel(
    _pool_sc,
    out_type=jax.ShapeDtypeStruct((_NC * _G, _D), jnp.float32),
    mesh=plsc.VectorSubcoreMesh(core_axis_name="c", subcore_axis_name="s"),
    compiler_params=pltpu.CompilerParams(use_tc_tiling_on_sc=True,
                                         needs_layout_passes=False),
    scratch_types=[
        pltpu.VMEM((_R, _D), jnp.float32),
        pltpu.VMEM((_R,), jnp.int32),
        pltpu.VMEM((_ACC,), jnp.float32),
        pltpu.VMEM((_D,), jnp.float32),
        pltpu.VMEM((_D,), jnp.float32),
        pltpu.VMEM_SHARED((_NS, 16 * _D), jnp.float32),
    ],
)


def _proj_body(part_ref, w_ref, b_ref, out_ref):
    pooled = part_ref[0] + part_ref[1]
    out_ref[...] = jax.lax.dot_general(
        pooled, w_ref[...],
        dimension_numbers=(((1,), (1,)), ((), ())),
        preferred_element_type=jnp.float32) + b_ref[...]


@jax.jit
def kernel(x_0, batch_0, W, b):
    partials = _pool(x_0, batch_0)
    logits = pl.pallas_call(
        _proj_body,
        out_shape=jax.ShapeDtypeStruct((_G, _D), jnp.float32),
    )(partials.reshape(_NC, _G, _D), W, b.reshape(1, _D))
    return logits


# speedup vs baseline: 1.6805x; 1.6805x over previous
import functools

import jax
import jax.numpy as jnp
from jax import lax
from jax.experimental import pallas as pl
from jax.experimental.pallas import tpu as pltpu
from jax.experimental.pallas import tpu_sc as plsc

_N = 50000
_D = 512
_G = 128
_R = 32
_NB = (_N + _R - 1) // _R
_LAST_BASE = _N - _R
_NC = 2
_NS = 16
_NW = _NC * _NS
_TRIPS = (_NB + _NW - 1) // _NW
_PAIRS = (_TRIPS + 2) // 2
_ACC = _G * _D


def _pool_sc(x_hbm, batch_hbm, part_hbm, rows0_v, rows1_v, idx0_v, idx1_v,
             accf_v, gat_v, red_v, stage_sh, sx0, sx1, si0, si1):
    cid = lax.axis_index("c")
    sid = lax.axis_index("s")
    wid = cid * _NS + sid

    rows_bufs = (rows0_v, rows1_v)
    idx_bufs = (idx0_v, idx1_v)
    x_sems = (sx0, sx1)
    i_sems = (si0, si1)

    def _zbody(i, carry):
        for k in range(8):
            accf_v[pl.ds(i * 128 + k * 16, 16)] = jnp.zeros((16,),
                                                            jnp.float32)
        return carry

    lax.fori_loop(0, _ACC // 128, _zbody, 0)

    lanes = lax.broadcasted_iota(jnp.int32, (16,), 0)

    def _base_of(t):
        blk = wid + t * _NW
        row_start = jnp.minimum(blk, _NB - 1) * _R
        return jnp.minimum(row_start, _LAST_BASE), blk

    def _start(t, b):
        base, _ = _base_of(t)
        pltpu.make_async_copy(x_hbm.at[pl.ds(base, _R)], rows_bufs[b],
                              x_sems[b]).start()
        pltpu.make_async_copy(batch_hbm.at[pl.ds(base, _R)], idx_bufs[b],
                              i_sems[b]).start()

    def _wait(t, b):
        base, _ = _base_of(t)
        pltpu.make_async_copy(x_hbm.at[pl.ds(base, _R)], rows_bufs[b],
                              x_sems[b]).wait()
        pltpu.make_async_copy(batch_hbm.at[pl.ds(base, _R)], idx_bufs[b],
                              i_sems[b]).wait()

    def _compute(t, b):
        rows_v = rows_bufs[b]
        idx_v = idx_bufs[b]
        base, blk = _base_of(t)

        @pl.when(blk < _NB)
        def _():
            shift = jnp.minimum(blk, _NB - 1) * _R - base

            def _grp(g, carry2):
                @pl.when((g * 16) >= shift)
                def _do():
                    ids16 = idx_v[pl.ds(g * 16, 16)]
                    first = ids16[0]
                    last = ids16[15]

                    @pl.when(first == last)
                    def _fast():
                        seg_vec = plsc.load_gather(
                            idx_v, [jnp.full((16,), g * 16, jnp.int32)])
                        tgt = seg_vec * _D + lanes

                        @plsc.parallel_loop(0, _D, 16, unroll=4)
                        def _strip(c):
                            vs = [rows_v[g * 16 + rr, pl.ds(c, 16)]
                                  for rr in range(16)]
                            while len(vs) > 1:
                                vs = [vs[i] + vs[i + 1]
                                      for i in range(0, len(vs) - 1, 2)] + (
                                          [vs[-1]] if len(vs) % 2 else [])
                            plsc.addupdate_scatter(accf_v, [tgt + c], vs[0])

                    @pl.when(first != last)
                    def _slow():
                        for rr in range(16):
                            seg_vec = plsc.load_gather(
                                idx_v,
                                [jnp.full((16,), g * 16 + rr, jnp.int32)])
                            tgt = seg_vec * _D + lanes

                            @plsc.parallel_loop(0, _D, 16, unroll=8)
                            def _strip(c):
                                vals = rows_v[g * 16 + rr, pl.ds(c, 16)]
                                plsc.addupdate_scatter(accf_v, [tgt + c],
                                                       vals)

                return carry2

            lax.fori_loop(0, _R // 16, _grp, 0)

    _start(0, 0)

    def _pair(p, carry):
        for b in range(2):
            t = 2 * p + b
            _wait(t, b)
            _start(t + 1, 1 - b)
            _compute(t, b)
        return carry

    lax.fori_loop(0, _PAIRS, _pair, 0)
    _wait(2 * _PAIRS, 0)

    def _round(q, carry):
        pltpu.sync_copy(accf_v.at[pl.ds(q * 16 * _D, 16 * _D)],
                        stage_sh.at[sid])
        plsc.subcore_barrier()

        pltpu.sync_copy(stage_sh.at[:, pl.ds(sid * _D, _D)], gat_v)

        def _abody(i, carry2):
            o = i * 16
            vs = [gat_v[src, pl.ds(o, 16)] for src in range(_NS)]
            while len(vs) > 1:
                vs = [vs[j] + vs[j + 1]
                      for j in range(0, len(vs) - 1, 2)] + (
                          [vs[-1]] if len(vs) % 2 else [])
            red_v[pl.ds(o, 16)] = vs[0]
            return carry2

        lax.fori_loop(0, _D // 16, _abody, 0)

        pltpu.sync_copy(red_v, part_hbm.at[cid * _G + q * 16 + sid])
        plsc.subcore_barrier()
        return carry

    lax.fori_loop(0, _G // 16, _round, 0)


_pool = pl.kernel(
    _pool_sc,
    out_type=jax.ShapeDtypeStruct((_NC * _G, _D), jnp.float32),
    mesh=plsc.VectorSubcoreMesh(core_axis_name="c", subcore_axis_name="s"),
    compiler_params=pltpu.CompilerParams(use_tc_tiling_on_sc=True,
                                         needs_layout_passes=False),
    scratch_types=[
        pltpu.VMEM((_R, _D), jnp.float32),
        pltpu.VMEM((_R, _D), jnp.float32),
        pltpu.VMEM((_R,), jnp.int32),
        pltpu.VMEM((_R,), jnp.int32),
        pltpu.VMEM((_ACC,), jnp.float32),
        pltpu.VMEM((_NS, _D), jnp.float32),
        pltpu.VMEM((_D,), jnp.float32),
        pltpu.VMEM_SHARED((_NS, 16 * _D), jnp.float32),
        pltpu.SemaphoreType.DMA,
        pltpu.SemaphoreType.DMA,
        pltpu.SemaphoreType.DMA,
        pltpu.SemaphoreType.DMA,
    ],
)


def _proj_body(part_ref, w_ref, b_ref, out_ref):
    pooled = part_ref[0] + part_ref[1]
    out_ref[...] = jax.lax.dot_general(
        pooled, w_ref[...],
        dimension_numbers=(((1,), (1,)), ((), ())),
        preferred_element_type=jnp.float32) + b_ref[...]


@jax.jit
def kernel(x_0, batch_0, W, b):
    partials = _pool(x_0, batch_0)
    logits = pl.pallas_call(
        _proj_body,
        out_shape=jax.ShapeDtypeStruct((_G, _D), jnp.float32),
    )(partials.reshape(_NC, _G, _D), W, b.reshape(1, _D))
    return logits


# speedup vs baseline: 2.4651x; 1.4669x over previous
import functools

import jax
import jax.numpy as jnp
from jax import lax
from jax.experimental import pallas as pl
from jax.experimental.pallas import tpu as pltpu
from jax.experimental.pallas import tpu_sc as plsc

_N = 50000
_D = 512
_G = 128
_N_SC = 12000
_BT = 2000
_TC_OFF = _N_SC // _BT
_NBT = (_N - _N_SC) // _BT
_R = 32
_NB = (_N_SC + _R - 1) // _R
_LAST_BASE = _N_SC - _R
_NC = 2
_NS = 16
_NW = _NC * _NS
_TRIPS = (_NB + _NW - 1) // _NW
_PAIRS = (_TRIPS + 2) // 2
_ACC = _G * _D


def _pool_sc(x_hbm, batch_hbm, part_hbm, rows0_v, rows1_v, idx0_v, idx1_v,
             accf_v, gat_v, red_v, stage_sh, sx0, sx1, si0, si1):
    cid = lax.axis_index("c")
    sid = lax.axis_index("s")
    wid = cid * _NS + sid

    rows_bufs = (rows0_v, rows1_v)
    idx_bufs = (idx0_v, idx1_v)
    x_sems = (sx0, sx1)
    i_sems = (si0, si1)

    def _zbody(i, carry):
        for k in range(8):
            accf_v[pl.ds(i * 128 + k * 16, 16)] = jnp.zeros((16,),
                                                            jnp.float32)
        return carry

    lax.fori_loop(0, _ACC // 128, _zbody, 0)

    lanes = lax.broadcasted_iota(jnp.int32, (16,), 0)

    def _base_of(t):
        blk = wid + t * _NW
        row_start = jnp.minimum(blk, _NB - 1) * _R
        return jnp.minimum(row_start, _LAST_BASE), blk

    def _start(t, b):
        base, _ = _base_of(t)
        pltpu.make_async_copy(x_hbm.at[pl.ds(base, _R)], rows_bufs[b],
                              x_sems[b]).start()
        pltpu.make_async_copy(batch_hbm.at[pl.ds(base, _R)], idx_bufs[b],
                              i_sems[b]).start()

    def _wait(t, b):
        base, _ = _base_of(t)
        pltpu.make_async_copy(x_hbm.at[pl.ds(base, _R)], rows_bufs[b],
                              x_sems[b]).wait()
        pltpu.make_async_copy(batch_hbm.at[pl.ds(base, _R)], idx_bufs[b],
                              i_sems[b]).wait()

    def _compute(t, b):
        rows_v = rows_bufs[b]
        idx_v = idx_bufs[b]
        base, blk = _base_of(t)

        @pl.when(blk < _NB)
        def _():
            shift = jnp.minimum(blk, _NB - 1) * _R - base

            def _grp(g, carry2):
                @pl.when((g * 16) >= shift)
                def _do():
                    ids16 = idx_v[pl.ds(g * 16, 16)]
                    first = ids16[0]
                    last = ids16[15]

                    @pl.when(first == last)
                    def _fast():
                        seg_vec = plsc.load_gather(
                            idx_v, [jnp.full((16,), g * 16, jnp.int32)])
                        tgt = seg_vec * _D + lanes

                        @plsc.parallel_loop(0, _D, 16, unroll=4)
                        def _strip(c):
                            vs = [rows_v[g * 16 + rr, pl.ds(c, 16)]
                                  for rr in range(16)]
                            while len(vs) > 1:
                                vs = [vs[i] + vs[i + 1]
                                      for i in range(0, len(vs) - 1, 2)] + (
                                          [vs[-1]] if len(vs) % 2 else [])
                            plsc.addupdate_scatter(accf_v, [tgt + c], vs[0])

                    @pl.when(first != last)
                    def _slow():
                        for rr in range(16):
                            seg_vec = plsc.load_gather(
                                idx_v,
                                [jnp.full((16,), g * 16 + rr, jnp.int32)])
                            tgt = seg_vec * _D + lanes

                            @plsc.parallel_loop(0, _D, 16, unroll=8)
                            def _strip(c):
                                vals = rows_v[g * 16 + rr, pl.ds(c, 16)]
                                plsc.addupdate_scatter(accf_v, [tgt + c],
                                                       vals)

                return carry2

            lax.fori_loop(0, _R // 16, _grp, 0)

    _start(0, 0)

    def _pair(p, carry):
        for b in range(2):
            t = 2 * p + b
            _wait(t, b)
            _start(t + 1, 1 - b)
            _compute(t, b)
        return carry

    lax.fori_loop(0, _PAIRS, _pair, 0)
    _wait(2 * _PAIRS, 0)

    def _round(q, carry):
        pltpu.sync_copy(accf_v.at[pl.ds(q * 16 * _D, 16 * _D)],
                        stage_sh.at[sid])
        plsc.subcore_barrier()

        pltpu.sync_copy(stage_sh.at[:, pl.ds(sid * _D, _D)], gat_v)

        def _abody(i, carry2):
            o = i * 16
            vs = [gat_v[src, pl.ds(o, 16)] for src in range(_NS)]
            while len(vs) > 1:
                vs = [vs[j] + vs[j + 1]
                      for j in range(0, len(vs) - 1, 2)] + (
                          [vs[-1]] if len(vs) % 2 else [])
            red_v[pl.ds(o, 16)] = vs[0]
            return carry2

        lax.fori_loop(0, _D // 16, _abody, 0)

        pltpu.sync_copy(red_v, part_hbm.at[cid * _G + q * 16 + sid])
        plsc.subcore_barrier()
        return carry

    lax.fori_loop(0, _G // 16, _round, 0)


_pool = pl.kernel(
    _pool_sc,
    out_type=jax.ShapeDtypeStruct((_NC * _G, _D), jnp.float32),
    mesh=plsc.VectorSubcoreMesh(core_axis_name="c", subcore_axis_name="s"),
    compiler_params=pltpu.CompilerParams(use_tc_tiling_on_sc=True,
                                         needs_layout_passes=False),
    scratch_types=[
        pltpu.VMEM((_R, _D), jnp.float32),
        pltpu.VMEM((_R, _D), jnp.float32),
        pltpu.VMEM((_R,), jnp.int32),
        pltpu.VMEM((_R,), jnp.int32),
        pltpu.VMEM((_ACC,), jnp.float32),
        pltpu.VMEM((_NS, _D), jnp.float32),
        pltpu.VMEM((_D,), jnp.float32),
        pltpu.VMEM_SHARED((_NS, 16 * _D), jnp.float32),
        pltpu.SemaphoreType.DMA,
        pltpu.SemaphoreType.DMA,
        pltpu.SemaphoreType.DMA,
        pltpu.SemaphoreType.DMA,
    ],
)


def _pool_tc_body(batch_ref, x_ref, out_ref):
    i = pl.program_id(0)
    ids = batch_ref[0, 0, :]
    seg = lax.broadcasted_iota(jnp.int32, (_G, _BT), 0)
    onehot = jnp.where(seg == ids[None, :], 1.0, 0.0).astype(jnp.float32)
    part = jax.lax.dot_general(
        onehot, x_ref[...],
        dimension_numbers=(((1,), (0,)), ((), ())),
        preferred_element_type=jnp.float32)

    @pl.when(i == 0)
    def _():
        out_ref[...] = part

    @pl.when(i != 0)
    def _():
        out_ref[...] += part


def _proj_body(part_ref, ptc_ref, w_ref, b_ref, out_ref):
    pooled = part_ref[0] + part_ref[1] + ptc_ref[...]
    out_ref[...] = jax.lax.dot_general(
        pooled, w_ref[...],
        dimension_numbers=(((1,), (1,)), ((), ())),
        preferred_element_type=jnp.float32) + b_ref[...]


@jax.jit
def kernel(x_0, batch_0, W, b):
    partials = _pool(x_0, batch_0)
    batch3 = batch_0.reshape(_N // _BT, 1, _BT)
    pooled_tc = pl.pallas_call(
        _pool_tc_body,
        grid=(_NBT,),
        in_specs=[
            pl.BlockSpec((1, 1, _BT), lambda i: (i + _TC_OFF, 0, 0)),
            pl.BlockSpec((_BT, _D), lambda i: (i + _TC_OFF, 0)),
        ],
        out_specs=pl.BlockSpec((_G, _D), lambda i: (0, 0)),
        out_shape=jax.ShapeDtypeStruct((_G, _D), jnp.float32),
    )(batch3, x_0)
    logits = pl.pallas_call(
        _proj_body,
        out_shape=jax.ShapeDtypeStruct((_G, _D), jnp.float32),
    )(partials.reshape(_NC, _G, _D), pooled_tc, W, b.reshape(1, _D))
    return logits


# speedup vs baseline: 2.7175x; 1.1024x over previous
import functools

import jax
import jax.numpy as jnp
from jax import lax
from jax.experimental import pallas as pl
from jax.experimental.pallas import tpu as pltpu
from jax.experimental.pallas import tpu_sc as plsc

_N = 50000
_D = 512
_G = 128
_N_SC = 12000
_BT = 2000
_TC_OFF = _N_SC // _BT
_NBT = (_N - _N_SC) // _BT
_R = 32
_NB = (_N_SC + _R - 1) // _R
_LAST_BASE = _N_SC - _R
_NC = 2
_NS = 16
_NW = _NC * _NS
_TRIPS = (_NB + _NW - 1) // _NW
_PAIRS = (_TRIPS + 2) // 2
_ACC = _G * _D


def _pool_sc(x_hbm, batch_hbm, part_hbm, rows0_v, rows1_v, idx0_v, idx1_v,
             accf_v, gat_v, red_v, zro_v, stage_sh, sx0, sx1, si0, si1):
    cid = lax.axis_index("c")
    sid = lax.axis_index("s")
    wid = cid * _NS + sid

    rows_bufs = (rows0_v, rows1_v)
    idx_bufs = (idx0_v, idx1_v)
    x_sems = (sx0, sx1)
    i_sems = (si0, si1)

    pltpu.sync_copy(batch_hbm.at[pl.ds(0, 16)], idx0_v.at[pl.ds(0, 16)])
    pltpu.sync_copy(batch_hbm.at[pl.ds(_N_SC - 16, 16)],
                    idx0_v.at[pl.ds(16, 16)])
    lo = idx0_v[pl.ds(0, 16)][0]
    hi = idx0_v[pl.ds(16, 16)][15]
    qlo = lo // 16
    qhi = hi // 16

    def _z512(i, carry):
        zro_v[pl.ds(i * 16, 16)] = jnp.zeros((16,), jnp.float32)
        return carry

    lax.fori_loop(0, _D // 16, _z512, 0)

    def _zbody(r, carry):
        for k in range(_D // 16):
            accf_v[pl.ds(r * _D + k * 16, 16)] = jnp.zeros((16,),
                                                           jnp.float32)
        return carry

    lax.fori_loop(qlo * 16, (qhi + 1) * 16, _zbody, 0)

    lanes = lax.broadcasted_iota(jnp.int32, (16,), 0)

    def _base_of(t):
        blk = wid + t * _NW
        row_start = jnp.minimum(blk, _NB - 1) * _R
        return jnp.minimum(row_start, _LAST_BASE), blk

    def _start(t, b):
        base, _ = _base_of(t)
        pltpu.make_async_copy(x_hbm.at[pl.ds(base, _R)], rows_bufs[b],
                              x_sems[b]).start()
        pltpu.make_async_copy(batch_hbm.at[pl.ds(base, _R)], idx_bufs[b],
                              i_sems[b]).start()

    def _wait(t, b):
        base, _ = _base_of(t)
        pltpu.make_async_copy(x_hbm.at[pl.ds(base, _R)], rows_bufs[b],
                              x_sems[b]).wait()
        pltpu.make_async_copy(batch_hbm.at[pl.ds(base, _R)], idx_bufs[b],
                              i_sems[b]).wait()

    def _compute(t, b):
        rows_v = rows_bufs[b]
        idx_v = idx_bufs[b]
        base, blk = _base_of(t)

        @pl.when(blk < _NB)
        def _():
            shift = jnp.minimum(blk, _NB - 1) * _R - base

            def _grp(g, carry2):
                @pl.when((g * 16) >= shift)
                def _do():
                    ids16 = idx_v[pl.ds(g * 16, 16)]
                    first = ids16[0]
                    last = ids16[15]

                    @pl.when(first == last)
                    def _fast():
                        seg_vec = plsc.load_gather(
                            idx_v, [jnp.full((16,), g * 16, jnp.int32)])
                        tgt = seg_vec * _D + lanes

                        @plsc.parallel_loop(0, _D, 16, unroll=4)
                        def _strip(c):
                            vs = [rows_v[g * 16 + rr, pl.ds(c, 16)]
                                  for rr in range(16)]
                            while len(vs) > 1:
                                vs = [vs[i] + vs[i + 1]
                                      for i in range(0, len(vs) - 1, 2)] + (
                                          [vs[-1]] if len(vs) % 2 else [])
                            plsc.addupdate_scatter(accf_v, [tgt + c], vs[0])

                    @pl.when(first != last)
                    def _slow():
                        for rr in range(16):
                            seg_vec = plsc.load_gather(
                                idx_v,
                                [jnp.full((16,), g * 16 + rr, jnp.int32)])
                            tgt = seg_vec * _D + lanes

                            @plsc.parallel_loop(0, _D, 16, unroll=8)
                            def _strip(c):
                                vals = rows_v[g * 16 + rr, pl.ds(c, 16)]
                                plsc.addupdate_scatter(accf_v, [tgt + c],
                                                       vals)

                return carry2

            lax.fori_loop(0, _R // 16, _grp, 0)

    _start(0, 0)

    def _pair(p, carry):
        for b in range(2):
            t = 2 * p + b
            _wait(t, b)
            _start(t + 1, 1 - b)
            _compute(t, b)
        return carry

    lax.fori_loop(0, _PAIRS, _pair, 0)
    _wait(2 * _PAIRS, 0)

    def _round(q, carry):
        active = jnp.logical_and(q >= qlo, q <= qhi)

        @pl.when(active)
        def _():
            pltpu.sync_copy(accf_v.at[pl.ds(q * 16 * _D, 16 * _D)],
                            stage_sh.at[sid])

        plsc.subcore_barrier()

        @pl.when(active)
        def _():
            pltpu.sync_copy(stage_sh.at[:, pl.ds(sid * _D, _D)], gat_v)

            def _abody(i, carry2):
                o = i * 16
                vs = [gat_v[src, pl.ds(o, 16)] for src in range(_NS)]
                while len(vs) > 1:
                    vs = [vs[j] + vs[j + 1]
                          for j in range(0, len(vs) - 1, 2)] + (
                              [vs[-1]] if len(vs) % 2 else [])
                red_v[pl.ds(o, 16)] = vs[0]
                return carry2

            lax.fori_loop(0, _D // 16, _abody, 0)

            pltpu.sync_copy(red_v, part_hbm.at[cid * _G + q * 16 + sid])

        @pl.when(jnp.logical_not(active))
        def _():
            pltpu.sync_copy(zro_v, part_hbm.at[cid * _G + q * 16 + sid])

        plsc.subcore_barrier()
        return carry

    lax.fori_loop(0, _G // 16, _round, 0)


_pool = pl.kernel(
    _pool_sc,
    out_type=jax.ShapeDtypeStruct((_NC * _G, _D), jnp.float32),
    mesh=plsc.VectorSubcoreMesh(core_axis_name="c", subcore_axis_name="s"),
    compiler_params=pltpu.CompilerParams(use_tc_tiling_on_sc=True,
                                         needs_layout_passes=False),
    scratch_types=[
        pltpu.VMEM((_R, _D), jnp.float32),
        pltpu.VMEM((_R, _D), jnp.float32),
        pltpu.VMEM((_R,), jnp.int32),
        pltpu.VMEM((_R,), jnp.int32),
        pltpu.VMEM((_ACC,), jnp.float32),
        pltpu.VMEM((_NS, _D), jnp.float32),
        pltpu.VMEM((_D,), jnp.float32),
        pltpu.VMEM((_D,), jnp.float32),
        pltpu.VMEM_SHARED((_NS, 16 * _D), jnp.float32),
        pltpu.SemaphoreType.DMA,
        pltpu.SemaphoreType.DMA,
        pltpu.SemaphoreType.DMA,
        pltpu.SemaphoreType.DMA,
    ],
)


def _pool_tc_body(batch_ref, x_ref, out_ref):
    i = pl.program_id(0)
    ids = batch_ref[0, 0, :]
    seg = lax.broadcasted_iota(jnp.int32, (_G, _BT), 0)
    onehot = jnp.where(seg == ids[None, :], 1.0, 0.0).astype(jnp.float32)
    part = jax.lax.dot_general(
        onehot, x_ref[...],
        dimension_numbers=(((1,), (0,)), ((), ())),
        preferred_element_type=jnp.float32)

    @pl.when(i == 0)
    def _():
        out_ref[...] = part

    @pl.when(i != 0)
    def _():
        out_ref[...] += part


def _proj_body(part_ref, ptc_ref, w_ref, b_ref, out_ref):
    pooled = part_ref[0] + part_ref[1] + ptc_ref[...]
    out_ref[...] = jax.lax.dot_general(
        pooled, w_ref[...],
        dimension_numbers=(((1,), (1,)), ((), ())),
        preferred_element_type=jnp.float32) + b_ref[...]


@jax.jit
def kernel(x_0, batch_0, W, b):
    partials = _pool(x_0, batch_0)
    batch3 = batch_0.reshape(_N // _BT, 1, _BT)
    pooled_tc = pl.pallas_call(
        _pool_tc_body,
        grid=(_NBT,),
        in_specs=[
            pl.BlockSpec((1, 1, _BT), lambda i: (i + _TC_OFF, 0, 0)),
            pl.BlockSpec((_BT, _D), lambda i: (i + _TC_OFF, 0)),
        ],
        out_specs=pl.BlockSpec((_G, _D), lambda i: (0, 0)),
        out_shape=jax.ShapeDtypeStruct((_G, _D), jnp.float32),
    )(batch3, x_0)
    logits = pl.pallas_call(
        _proj_body,
        out_shape=jax.ShapeDtypeStruct((_G, _D), jnp.float32),
    )(partials.reshape(_NC, _G, _D), pooled_tc, W, b.reshape(1, _D))
    return logits
